# baseline (speedup 1.0000x reference)
import functools

import jax
import jax.numpy as jnp
from jax import lax
from jax.experimental import pallas as pl
from jax.experimental.pallas import tpu as pltpu
from jax.experimental.pallas import tpu_sc as plsc

N = 10000
F = 128
NC, NS = 2, 16
NW = NC * NS
NPAD = 10240
RPW = NPAD // NS
CH = 80
E = 320000
EPW = E // NW
CPW = EPW // CH

_MESH = plsc.VectorSubcoreMesh(core_axis_name="c", subcore_axis_name="s")


DQ = 8


def _deg_body(dst_hbm, out_hbm, dst_v, ones_v, zvec_v, cnt_sh, ssem):
    c = lax.axis_index("c")
    s = lax.axis_index("s")
    w = s * NC + c
    one16 = jnp.full((16,), 1.0, jnp.float32)
    zero16 = jnp.zeros((16,), jnp.float32)
    for k in range(CH // 16):
        ones_v[pl.ds(k * 16, 16)] = one16

    def zb(i, carry):
        zvec_v[pl.ds(i * 16, 16)] = zero16
        return carry

    lax.fori_loop(0, RPW // 16, zb, 0)
    pltpu.sync_copy(zvec_v, cnt_sh.at[pl.ds(s * RPW, RPW)])
    pltpu.sync_copy(dst_hbm.at[w], dst_v)
    plsc.subcore_barrier()

    for k in range(DQ):
        pltpu.async_copy(ones_v, cnt_sh.at[dst_v.at[k]], ssem, add=True)

    def body(j, carry):
        pltpu.make_async_copy(ones_v, cnt_sh.at[dst_v.at[0]], ssem).wait()
        pltpu.async_copy(ones_v, cnt_sh.at[dst_v.at[j + DQ]], ssem, add=True)
        return carry

    lax.fori_loop(0, CPW - DQ, body, 0)
    for k in range(DQ):
        pltpu.make_async_copy(ones_v, cnt_sh.at[dst_v.at[0]], ssem).wait()
    plsc.subcore_barrier()
    pltpu.sync_copy(cnt_sh.at[pl.ds(s * RPW, RPW)],
                    out_hbm.at[c, pl.ds(s * RPW, RPW)])


_deg = pl.kernel(
    _deg_body,
    mesh=_MESH,
    out_type=jax.ShapeDtypeStruct((NC, NPAD), jnp.float32),
    scratch_types=[
        pltpu.VMEM((CPW, CH), jnp.int32),
        pltpu.VMEM((CH,), jnp.float32),
        pltpu.VMEM((RPW,), jnp.float32),
        pltpu.VMEM_SHARED((NPAD,), jnp.float32),
        pltpu.SemaphoreType.DMA,
    ],
)


NB = 4
G = 5
NGB = CPW // G


def _agg_body(tab_hbm, edg_hbm, out_hbm, gb0, gb1, msg0, msg1, msg2, msg3,
              agg_sh, isemA, isemB, gsem0, gsem1, gsem2, gsem3):
    gbufs = (gb0, gb1)
    msgs = (msg0, msg1, msg2, msg3)
    isems = (isemA, isemB)
    gsems = (gsem0, gsem1, gsem2, gsem3)
    c = lax.axis_index("c")
    s = lax.axis_index("s")
    w = s * NC + c
    zero16 = jnp.zeros((16,), jnp.float32)

    def zrow(i, carry):
        for k in range(F // 16):
            msgs[0][i, pl.ds(k * 16, 16)] = zero16
        return carry

    lax.fori_loop(0, CH, zrow, 0)
    for t in range(RPW // CH):
        pltpu.sync_copy(msgs[0], agg_sh.at[pl.ds(s * RPW + t * CH, CH)])
    plsc.subcore_barrier()

    pltpu.sync_copy(edg_hbm.at[w, 0], gbufs[0])
    pltpu.async_copy(edg_hbm.at[w, 1], gbufs[1], isems[1])
    for k in range(NB - 1):
        pltpu.async_copy(tab_hbm.at[gbufs[0].at[k, 0]], msgs[k], gsems[k])

    def group_step(gb_dyn, g, reload):
        b = g % 2
        for u in range(G):
            sl = (5 * g + u) % NB
            if u == 2:
                pltpu.make_async_copy(edg_hbm.at[w, 0], gbufs[(g + 1) % 2],
                                      isems[(g + 1) % 2]).wait()
            pltpu.make_async_copy(tab_hbm.at[gbufs[b].at[u, 0]], msgs[sl],
                                  gsems[sl]).wait()
            pltpu.sync_copy(msgs[sl], agg_sh.at[gbufs[b].at[u, 1]], add=True)
            r = (u + NB - 1) % G
            bb = (g + (1 if u + NB - 1 >= G else 0)) % 2
            sn = (5 * g + u + NB - 1) % NB
            pltpu.async_copy(tab_hbm.at[gbufs[bb].at[r, 0]], msgs[sn],
                             gsems[sn])
        if reload:
            pltpu.async_copy(edg_hbm.at[w, (gb_dyn + 2) % NGB], gbufs[b],
                             isems[b])

    def body(t, carry):
        for g in range(4):
            group_step(4 * t + g, g, True)
        return carry

    lax.fori_loop(0, NGB // 4, body, 0)
    group_step(NGB - 1, 0, False)
    for i in range(1, NB):
        k = (CPW - 1 + i) % NB
        pltpu.make_async_copy(tab_hbm.at[gbufs[0].at[0, 0]], msgs[k],
                              gsems[k]).wait()
    plsc.subcore_barrier()
    pltpu.sync_copy(agg_sh.at[pl.ds(s * RPW, RPW)],
                    out_hbm.at[c, pl.ds(s * RPW, RPW)])


_agg = pl.kernel(
    _agg_body,
    mesh=_MESH,
    out_type=jax.ShapeDtypeStruct((NC, NPAD, F), jnp.float32),
    scratch_types=(
        [pltpu.VMEM((G, 2, CH), jnp.int32) for _ in range(2)]
        + [pltpu.VMEM((CH, F), jnp.float32) for _ in range(NB)]
        + [pltpu.VMEM_SHARED((NPAD, F), jnp.float32)]
        + [pltpu.SemaphoreType.DMA for _ in range(2 + NB)]
    ),
)

_RB = 1000


def _tc1_body(cnt_ref, x_ref, dinv_ref, xs_ref):
    cnt = cnt_ref[...]
    d = lax.rsqrt(cnt[:, 0:1] + cnt[:, 1:2] + 1.0)
    dinv_ref[...] = d
    xs_ref[...] = x_ref[...] * d


def _tc1(cnt_col, x):
    return pl.pallas_call(
        _tc1_body,
        grid=(N // _RB,),
        in_specs=[
            pl.BlockSpec((_RB, 2), lambda i: (i, 0)),
            pl.BlockSpec((_RB, F), lambda i: (i, 0)),
        ],
        out_specs=[
            pl.BlockSpec((_RB, 1), lambda i: (i, 0)),
            pl.BlockSpec((_RB, F), lambda i: (i, 0)),
        ],
        out_shape=[
            jax.ShapeDtypeStruct((N, 1), jnp.float32),
            jax.ShapeDtypeStruct((NPAD, F), jnp.float32),
        ],
    )(cnt_col, x)


def _tc2_body(dinv_ref, xs_ref, agg_ref, W1_ref, b1_ref, W2_ref, h2s_ref):
    d = dinv_ref[...]
    s1 = d * (agg_ref[0] + agg_ref[1] + xs_ref[...])
    h1 = jnp.dot(s1, W1_ref[...], preferred_element_type=jnp.float32)
    h1 = jnp.maximum(h1 + b1_ref[...], 0.0)
    h2s_ref[...] = d * jnp.dot(h1, W2_ref[...],
                               preferred_element_type=jnp.float32)


def _tc2(dinv, xs, agg1, W1, b1r, W2):
    return pl.pallas_call(
        _tc2_body,
        grid=(N // _RB,),
        in_specs=[
            pl.BlockSpec((_RB, 1), lambda i: (i, 0)),
            pl.BlockSpec((_RB, F), lambda i: (i, 0)),
            pl.BlockSpec((NC, _RB, F), lambda i: (0, i, 0)),
            pl.BlockSpec((F, 256), lambda i: (0, 0)),
            pl.BlockSpec((1, 256), lambda i: (0, 0)),
            pl.BlockSpec((256, F), lambda i: (0, 0)),
        ],
        out_specs=pl.BlockSpec((_RB, F), lambda i: (i, 0)),
        out_shape=jax.ShapeDtypeStruct((NPAD, F), jnp.float32),
    )(dinv, xs, agg1, W1, b1r, W2)


def _tc3_body(dinv_ref, h2s_ref, agg_ref, b2_ref, W3_ref, b3_ref, out_ref):
    d = dinv_ref[...]
    s2 = d * (agg_ref[0] + agg_ref[1] + h2s_ref[...]) + b2_ref[...]
    h2 = jnp.where(s2 >= 0, s2, 0.01 * s2)
    out_ref[...] = jnp.dot(h2, W3_ref[...],
                           preferred_element_type=jnp.float32) + b3_ref[...]


def _tc3(dinv, h2s, agg2, b2r, W3p, b3p):
    return pl.pallas_call(
        _tc3_body,
        grid=(N // _RB,),
        in_specs=[
            pl.BlockSpec((_RB, 1), lambda i: (i, 0)),
            pl.BlockSpec((_RB, F), lambda i: (i, 0)),
            pl.BlockSpec((NC, _RB, F), lambda i: (0, i, 0)),
            pl.BlockSpec((1, F), lambda i: (0, 0)),
            pl.BlockSpec((F, 8), lambda i: (0, 0)),
            pl.BlockSpec((1, 8), lambda i: (0, 0)),
        ],
        out_specs=pl.BlockSpec((_RB, 8), lambda i: (i, 0)),
        out_shape=jax.ShapeDtypeStruct((N, 8), jnp.float32),
    )(dinv, h2s, agg2, b2r, W3p, b3p)


def kernel(x, edge_index, W1, b1, W2, b2, W3, b3):
    ei = edge_index.astype(jnp.int32)
    dst3 = ei[1].reshape(NW, CPW, CH)
    edg = jnp.concatenate(
        [ei[0].reshape(NW, NGB, G, 1, CH), ei[1].reshape(NW, NGB, G, 1, CH)],
        axis=3)

    cnt2 = _deg(dst3)
    cnt_col = cnt2[:, :N].T
    dinv, xs = _tc1(cnt_col, x)

    agg1 = _agg(xs, edg)
    h2s = _tc2(dinv, xs, agg1, W1, b1.reshape(1, -1), W2)

    agg2 = _agg(h2s, edg)
    W3p = jnp.pad(W3, ((0, 0), (0, 1)))
    b3p = jnp.pad(b3, (0, 1)).reshape(1, -1)
    out8 = _tc3(dinv, h2s, agg2, b2.reshape(1, -1), W3p, b3p)
    return out8[:, :7]


# speedup vs baseline: 1.0342x; 1.0342x over previous
import jax
import jax.numpy as jnp
from jax import lax
from jax.experimental import pallas as pl
from jax.experimental.pallas import tpu as pltpu
from jax.experimental.pallas import tpu_sc as plsc

N = 10000
F = 128
NC, NS = 2, 16
NW = NC * NS
NPAD = 10240
RPW = NPAD // NS
CH = 80
E = 320000
EPW = E // NW
CPW = EPW // CH

_MESH = plsc.VectorSubcoreMesh(core_axis_name="c", subcore_axis_name="s")


DQ = 8


def _deg_body(dst_hbm, out_hbm, dst_v, ones_v, zvec_v, cnt_sh, ssem):
    c = lax.axis_index("c")
    s = lax.axis_index("s")
    w = s * NC + c
    one16 = jnp.full((16,), 1.0, jnp.float32)
    zero16 = jnp.zeros((16,), jnp.float32)
    for k in range(CH // 16):
        ones_v[pl.ds(k * 16, 16)] = one16

    def zb(i, carry):
        zvec_v[pl.ds(i * 16, 16)] = zero16
        return carry

    lax.fori_loop(0, RPW // 16, zb, 0)
    pltpu.sync_copy(zvec_v, cnt_sh.at[pl.ds(s * RPW, RPW)])
    pltpu.sync_copy(dst_hbm.at[w], dst_v)
    plsc.subcore_barrier()

    for k in range(DQ):
        pltpu.async_copy(ones_v, cnt_sh.at[dst_v.at[k]], ssem, add=True)

    def body(j, carry):
        pltpu.make_async_copy(ones_v, cnt_sh.at[dst_v.at[0]], ssem).wait()
        pltpu.async_copy(ones_v, cnt_sh.at[dst_v.at[j + DQ]], ssem, add=True)
        return carry

    lax.fori_loop(0, CPW - DQ, body, 0)
    for k in range(DQ):
        pltpu.make_async_copy(ones_v, cnt_sh.at[dst_v.at[0]], ssem).wait()
    plsc.subcore_barrier()
    pltpu.sync_copy(cnt_sh.at[pl.ds(s * RPW, RPW)],
                    out_hbm.at[c, pl.ds(s * RPW, RPW)])


_deg = pl.kernel(
    _deg_body,
    mesh=_MESH,
    out_type=jax.ShapeDtypeStruct((NC, NPAD), jnp.float32),
    scratch_types=[
        pltpu.VMEM((CPW, CH), jnp.int32),
        pltpu.VMEM((CH,), jnp.float32),
        pltpu.VMEM((RPW,), jnp.float32),
        pltpu.VMEM_SHARED((NPAD,), jnp.float32),
        pltpu.SemaphoreType.DMA,
    ],
)


NB = 4


def _agg_body(tab_hbm, src_hbm, dst_hbm, out_hbm, *rest):
    svs = rest[:NB]
    dvs = rest[NB:2 * NB]
    msgs = rest[2 * NB:3 * NB]
    agg_sh = rest[3 * NB]
    isems = rest[3 * NB + 1:3 * NB + 1 + NB]
    gsems = rest[3 * NB + 1 + NB:]
    c = lax.axis_index("c")
    s = lax.axis_index("s")
    w = s * NC + c
    base = w * EPW
    zero16 = jnp.zeros((16,), jnp.float32)

    def zrow(i, carry):
        for k in range(F // 16):
            msgs[0][i, pl.ds(k * 16, 16)] = zero16
        return carry

    lax.fori_loop(0, CH, zrow, 0)
    for t in range(RPW // CH):
        pltpu.sync_copy(msgs[0], agg_sh.at[pl.ds(s * RPW + t * CH, CH)])
    plsc.subcore_barrier()

    def load_idx(j, k, sem):
        pltpu.async_copy(src_hbm.at[pl.ds(base + j * CH, CH)], svs[k], sem)
        pltpu.async_copy(dst_hbm.at[pl.ds(base + j * CH, CH)], dvs[k], sem)

    def wait_idx(k, sem):
        pltpu.make_async_copy(src_hbm.at[pl.ds(0, CH)], svs[k], sem).wait()
        pltpu.make_async_copy(dst_hbm.at[pl.ds(0, CH)], dvs[k], sem).wait()

    for k in range(NB - 1):
        pltpu.sync_copy(src_hbm.at[pl.ds(base + k * CH, CH)], svs[k])
        pltpu.sync_copy(dst_hbm.at[pl.ds(base + k * CH, CH)], dvs[k])
    load_idx(NB - 1, NB - 1, isems[NB - 1])
    for k in range(NB - 1):
        pltpu.async_copy(tab_hbm.at[svs[k]], msgs[k], gsems[k])

    def chunk_step(j, u):
        p = (u + NB - 1) % NB
        pltpu.make_async_copy(tab_hbm.at[svs[u]], msgs[u], gsems[u]).wait()
        pltpu.sync_copy(msgs[u], agg_sh.at[dvs[u]], add=True)
        load_idx((j + NB) % CPW, u, isems[u])
        wait_idx(p, isems[p])
        pltpu.async_copy(tab_hbm.at[svs[p]], msgs[p], gsems[p])

    def body(t, carry):
        for u in range(NB):
            chunk_step(NB * t + u, u)
        return carry

    nfull = CPW // NB
    lax.fori_loop(0, nfull, body, 0)
    for u in range(CPW % NB):
        chunk_step(NB * nfull + u, u)
    for i in range(1, NB):
        k = (CPW - 1 + i) % NB
        pltpu.make_async_copy(tab_hbm.at[svs[k]], msgs[k], gsems[k]).wait()
    wait_idx((CPW - 1) % NB, isems[(CPW - 1) % NB])
    plsc.subcore_barrier()
    pltpu.sync_copy(agg_sh.at[pl.ds(s * RPW, RPW)],
                    out_hbm.at[c, pl.ds(s * RPW, RPW)])


_agg = pl.kernel(
    _agg_body,
    mesh=_MESH,
    out_type=jax.ShapeDtypeStruct((NC, NPAD, F), jnp.float32),
    scratch_types=(
        [pltpu.VMEM((CH,), jnp.int32) for _ in range(2 * NB)]
        + [pltpu.VMEM((CH, F), jnp.float32) for _ in range(NB)]
        + [pltpu.VMEM_SHARED((NPAD, F), jnp.float32)]
        + [pltpu.SemaphoreType.DMA for _ in range(2 * NB)]
    ),
)

_RB = 1000


def _tc1_body(cnt_ref, x_ref, dinv_ref, xs_ref):
    cnt = cnt_ref[...]
    d = lax.rsqrt(cnt[:, 0:1] + cnt[:, 1:2] + 1.0)
    dinv_ref[...] = d
    xs_ref[...] = x_ref[...] * d


def _tc1(cnt_col, x):
    return pl.pallas_call(
        _tc1_body,
        grid=(N // _RB,),
        in_specs=[
            pl.BlockSpec((_RB, 2), lambda i: (i, 0)),
            pl.BlockSpec((_RB, F), lambda i: (i, 0)),
        ],
        out_specs=[
            pl.BlockSpec((_RB, 1), lambda i: (i, 0)),
            pl.BlockSpec((_RB, F), lambda i: (i, 0)),
        ],
        out_shape=[
            jax.ShapeDtypeStruct((N, 1), jnp.float32),
            jax.ShapeDtypeStruct((NPAD, F), jnp.float32),
        ],
    )(cnt_col, x)


def _tc2_body(dinv_ref, xs_ref, agg_ref, W1_ref, b1_ref, W2_ref, h2s_ref):
    d = dinv_ref[...]
    s1 = d * (agg_ref[0] + agg_ref[1] + xs_ref[...])
    h1 = jnp.dot(s1, W1_ref[...], preferred_element_type=jnp.float32)
    h1 = jnp.maximum(h1 + b1_ref[...], 0.0)
    h2s_ref[...] = d * jnp.dot(h1, W2_ref[...],
                               preferred_element_type=jnp.float32)


def _tc2(dinv, xs, agg1, W1, b1r, W2):
    return pl.pallas_call(
        _tc2_body,
        grid=(N // _RB,),
        in_specs=[
            pl.BlockSpec((_RB, 1), lambda i: (i, 0)),
            pl.BlockSpec((_RB, F), lambda i: (i, 0)),
            pl.BlockSpec((NC, _RB, F), lambda i: (0, i, 0)),
            pl.BlockSpec((F, 256), lambda i: (0, 0)),
            pl.BlockSpec((1, 256), lambda i: (0, 0)),
            pl.BlockSpec((256, F), lambda i: (0, 0)),
        ],
        out_specs=pl.BlockSpec((_RB, F), lambda i: (i, 0)),
        out_shape=jax.ShapeDtypeStruct((NPAD, F), jnp.float32),
    )(dinv, xs, agg1, W1, b1r, W2)


def _tc3_body(dinv_ref, h2s_ref, agg_ref, b2_ref, W3_ref, b3_ref, out_ref):
    d = dinv_ref[...]
    s2 = d * (agg_ref[0] + agg_ref[1] + h2s_ref[...]) + b2_ref[...]
    h2 = jnp.where(s2 >= 0, s2, 0.01 * s2)
    out_ref[...] = jnp.dot(h2, W3_ref[...],
                           preferred_element_type=jnp.float32) + b3_ref[...]


def _tc3(dinv, h2s, agg2, b2r, W3p, b3p):
    return pl.pallas_call(
        _tc3_body,
        grid=(N // _RB,),
        in_specs=[
            pl.BlockSpec((_RB, 1), lambda i: (i, 0)),
            pl.BlockSpec((_RB, F), lambda i: (i, 0)),
            pl.BlockSpec((NC, _RB, F), lambda i: (0, i, 0)),
            pl.BlockSpec((1, F), lambda i: (0, 0)),
            pl.BlockSpec((F, 8), lambda i: (0, 0)),
            pl.BlockSpec((1, 8), lambda i: (0, 0)),
        ],
        out_specs=pl.BlockSpec((_RB, 8), lambda i: (i, 0)),
        out_shape=jax.ShapeDtypeStruct((N, 8), jnp.float32),
    )(dinv, h2s, agg2, b2r, W3p, b3p)


def kernel(x, edge_index, W1, b1, W2, b2, W3, b3):
    ei = edge_index.astype(jnp.int32)
    dst3 = ei[1].reshape(NW, CPW, CH)

    cnt2 = _deg(dst3)
    cnt_col = cnt2[:, :N].T
    dinv, xs = _tc1(cnt_col, x)

    agg1 = _agg(xs, ei[0], ei[1])
    h2s = _tc2(dinv, xs, agg1, W1, b1.reshape(1, -1), W2)

    agg2 = _agg(h2s, ei[0], ei[1])
    W3p = jnp.pad(W3, ((0, 0), (0, 1)))
    b3p = jnp.pad(b3, (0, 1)).reshape(1, -1)
    out8 = _tc3(dinv, h2s, agg2, b2.reshape(1, -1), W3p, b3p)
    return out8[:, :7]
